# baseline (speedup 1.0000x reference)
import functools

import jax
import jax.numpy as jnp
from jax import lax
from jax.experimental import pallas as pl
from jax.experimental.pallas import tpu as pltpu
from jax.experimental.pallas import tpu_sc as plsc

N_NODES = 10000
N_EDGES = 320000
D = 128
N_REL = 6

NC = 2
NS = 16
NW = NC * NS

CHUNK = 64
GROUPS = CHUNK // 16
CHUNKS_PER_W = 158
EPW = CHUNKS_PER_W * CHUNK
E_PAD = NW * EPW
NLANE = 16
KBLK = D // NLANE
IDXB = 3 * CHUNK


def _make_sc_kernel():
    mesh = plsc.VectorSubcoreMesh(
        core_axis_name="c", subcore_axis_name="s",
        num_cores=NC, num_subcores=NS)

    kernel_wrap = functools.partial(
        pl.kernel,
        out_type=jax.ShapeDtypeStruct((E_PAD,), jnp.float32),
        mesh=mesh,
        scratch_types=[
            pltpu.VMEM((IDXB,), jnp.int32),
            pltpu.VMEM((IDXB,), jnp.int32),
            pltpu.VMEM((CHUNK, D), jnp.float32),
            pltpu.VMEM((CHUNK, D), jnp.float32),
            pltpu.VMEM((CHUNK, D), jnp.float32),
            pltpu.VMEM((CHUNK, D), jnp.float32),
            pltpu.VMEM((N_REL * D,), jnp.float32),
            pltpu.VMEM((NLANE, NLANE + 1), jnp.float32),
            pltpu.VMEM((CHUNK,), jnp.float32),
            pltpu.VMEM((CHUNK,), jnp.float32),
            pltpu.SemaphoreType.DMA,
            pltpu.SemaphoreType.DMA,
            pltpu.SemaphoreType.DMA,
            pltpu.SemaphoreType.DMA,
            pltpu.SemaphoreType.DMA,
            pltpu.SemaphoreType.DMA,
            pltpu.VMEM_SHARED((N_NODES, D), jnp.float32),
        ],
        compiler_params=pltpu.CompilerParams(needs_layout_passes=False),
    )

    def distmult(h_hbm, idx_hbm, w_hbm, out_hbm,
                 idx_a, idx_b, rows_sa, rows_ta, rows_sb, rows_tb,
                 w_v, tbuf, out_a, out_b,
                 sem_ra, sem_rb, sem_ia, sem_ib, sem_oa, sem_ob, h_sp):
        cid = lax.axis_index("c")
        sid = lax.axis_index("s")
        wid = sid * NC + cid
        base_w = wid * EPW
        idx_base = wid * CHUNKS_PER_W * IDXB

        pltpu.sync_copy(w_hbm, w_v)

        @pl.when(sid == 0)
        def _stage_h():
            pltpu.sync_copy(h_hbm, h_sp)
        plsc.subcore_barrier()

        def fire_idx(ci, idx_v, sem):
            pltpu.async_copy(
                idx_hbm.at[pl.ds(idx_base + ci * IDXB, IDXB)], idx_v, sem)

        def wait_idx(idx_v, sem):
            pltpu.make_async_copy(
                idx_hbm.at[pl.ds(0, IDXB)], idx_v, sem).wait()

        def fire_rows(idx_v, rows_s, rows_t, sem):
            pltpu.async_copy(h_sp.at[idx_v.at[pl.ds(0, CHUNK)]], rows_s, sem)
            pltpu.async_copy(h_sp.at[idx_v.at[pl.ds(CHUNK, CHUNK)]], rows_t,
                             sem)

        def wait_rows(idx_v, rows_s, rows_t, sem):
            pltpu.make_async_copy(h_sp.at[idx_v.at[pl.ds(0, CHUNK)]],
                                  rows_s, sem).wait()
            pltpu.make_async_copy(h_sp.at[idx_v.at[pl.ds(0, CHUNK)]],
                                  rows_t, sem).wait()

        def fire_out(ci, out_v, sem):
            pltpu.async_copy(out_v, out_hbm.at[pl.ds(base_w + ci * CHUNK,
                                                     CHUNK)], sem)

        def wait_out(out_v, sem):
            pltpu.make_async_copy(out_v, out_hbm.at[pl.ds(0, CHUNK)],
                                  sem).wait()

        iota16 = lax.iota(jnp.int32, NLANE)

        def compute(idx_v, rows_s, rows_t, out_v):
            def group_body(g, _):
                rel_g = idx_v[pl.ds(2 * CHUNK + g * NLANE, NLANE)]
                for e in range(NLANE):
                    r = g * NLANE + e
                    relsp = rel_g[jnp.full((NLANE,), e, jnp.int32)]
                    wbase = relsp * D + iota16
                    acc = None
                    for k in range(KBLK):
                        wk = plsc.load_gather(w_v, [wbase + (k * NLANE)])
                        p = (rows_s[r, pl.ds(k * NLANE, NLANE)] *
                             rows_t[r, pl.ds(k * NLANE, NLANE)]) * wk
                        acc = p if acc is None else acc + p
                    tbuf[e, pl.ds(0, NLANE)] = acc
                score = plsc.load_gather(
                    tbuf, [iota16, jnp.zeros((NLANE,), jnp.int32)])
                for k in range(1, NLANE):
                    score = score + plsc.load_gather(
                        tbuf, [iota16, jnp.full((NLANE,), k, jnp.int32)])
                out_v[pl.ds(g * NLANE, NLANE)] = (
                    1.0 / (1.0 + jnp.exp(-score)))
                return 0
            lax.fori_loop(0, GROUPS, group_body, 0)

        fire_idx(0, idx_a, sem_ia)
        fire_idx(1, idx_b, sem_ib)
        wait_idx(idx_a, sem_ia)
        fire_rows(idx_a, rows_sa, rows_ta, sem_ra)
        wait_idx(idx_b, sem_ib)
        fire_rows(idx_b, rows_sb, rows_tb, sem_rb)

        def chunk_pair(j, _):
            ca = 2 * j
            wait_rows(idx_a, rows_sa, rows_ta, sem_ra)

            @pl.when(j > 0)
            def _():
                wait_out(out_a, sem_oa)
            compute(idx_a, rows_sa, rows_ta, out_a)
            fire_out(ca, out_a, sem_oa)

            @pl.when(ca + 2 < CHUNKS_PER_W)
            def _():
                fire_idx(ca + 2, idx_a, sem_ia)

            wait_rows(idx_b, rows_sb, rows_tb, sem_rb)

            @pl.when(j > 0)
            def _():
                wait_out(out_b, sem_ob)
            compute(idx_b, rows_sb, rows_tb, out_b)
            fire_out(ca + 1, out_b, sem_ob)

            @pl.when(ca + 3 < CHUNKS_PER_W)
            def _():
                fire_idx(ca + 3, idx_b, sem_ib)

            @pl.when(ca + 2 < CHUNKS_PER_W)
            def _():
                wait_idx(idx_a, sem_ia)
                fire_rows(idx_a, rows_sa, rows_ta, sem_ra)

            @pl.when(ca + 3 < CHUNKS_PER_W)
            def _():
                wait_idx(idx_b, sem_ib)
                fire_rows(idx_b, rows_sb, rows_tb, sem_rb)
            return 0

        lax.fori_loop(0, CHUNKS_PER_W // 2, chunk_pair, 0)

        wait_out(out_a, sem_oa)
        wait_out(out_b, sem_ob)

    return kernel_wrap(distmult)


_DISTMULT = _make_sc_kernel()


def kernel(h, edge_index, rel_ids, W):
    src = edge_index[0].astype(jnp.int32)
    dst = edge_index[1].astype(jnp.int32)
    rel = rel_ids.astype(jnp.int32)
    pad = E_PAD - N_EDGES
    src = jnp.concatenate([src, jnp.zeros((pad,), jnp.int32)])
    dst = jnp.concatenate([dst, jnp.zeros((pad,), jnp.int32)])
    rel = jnp.concatenate([rel, jnp.zeros((pad,), jnp.int32)])
    packed = jnp.stack([src.reshape(-1, CHUNK), dst.reshape(-1, CHUNK),
                        rel.reshape(-1, CHUNK)], axis=1).reshape(-1)
    w_flat = W.reshape(-1)
    out = _DISTMULT(h, packed, w_flat)
    return out[:N_EDGES]
